# Initial kernel scaffold; baseline (speedup 1.0000x reference)
import jax
import jax.numpy as jnp
from jax.experimental import pallas as pl


def kernel(x, gate_w, gate_b, w2, w3, w1, deterministic):
    raise NotImplementedError("write your pallas kernel here")


# speedup vs baseline: 1.8414x; 1.8414x over previous
import functools

import jax
import jax.numpy as jnp
from jax import lax
from jax.experimental import pallas as pl
from jax.experimental.pallas import tpu as pltpu

DIM = 1024
HIDDEN = 2048
E = 8
EPAD = 128
TB = 128
CAPACITY_FACTOR = 1.25


def _routing_body(cap, x_ref, gw_ref, gb_ref, slot_ref, gwt_ref, carry_ref):
    i = pl.program_id(0)

    @pl.when(i == 0)
    def _():
        carry_ref[...] = jnp.zeros_like(carry_ref)

    x = x_ref[...]
    logits = jnp.dot(x, gw_ref[...], preferred_element_type=jnp.float32)
    logits = logits + gb_ref[...]
    lane = lax.broadcasted_iota(jnp.int32, (TB, EPAD), 1)
    logits = jnp.where(lane < E, logits, -1e30)
    m = jnp.max(logits, axis=1, keepdims=True)
    eid = jnp.min(jnp.where(logits >= m, lane, EPAD), axis=1, keepdims=True)
    denom = jnp.sum(jnp.where(lane < E, jnp.exp(logits - m), 0.0),
                    axis=1, keepdims=True)
    gw_tok = 1.0 / denom

    onehot = (lane == eid).astype(jnp.float32)
    row = lax.broadcasted_iota(jnp.int32, (TB, TB), 0)
    col = lax.broadcasted_iota(jnp.int32, (TB, TB), 1)
    ltri = (col <= row).astype(jnp.float32)
    within = jnp.dot(ltri, onehot, preferred_element_type=jnp.float32)
    carry = carry_ref[...]
    pos = jnp.sum((within + carry) * onehot, axis=1, keepdims=True)
    carry_ref[...] = carry + jnp.sum(onehot, axis=0, keepdims=True)

    valid = pos <= cap
    slot = jnp.where(valid, eid * cap + pos.astype(jnp.int32) - 1, E * cap)
    slot_ref[...] = slot
    gwt_ref[...] = jnp.where(valid, gw_tok, 0.0)


def _routing(xf, gwp, gbp, cap):
    t = xf.shape[0]
    nb = t // TB
    return pl.pallas_call(
        functools.partial(_routing_body, cap),
        grid=(nb,),
        in_specs=[
            pl.BlockSpec((TB, DIM), lambda i: (i, 0)),
            pl.BlockSpec((DIM, EPAD), lambda i: (0, 0)),
            pl.BlockSpec((1, EPAD), lambda i: (0, 0)),
        ],
        out_specs=[
            pl.BlockSpec((TB, 1), lambda i: (i, 0)),
            pl.BlockSpec((TB, 1), lambda i: (i, 0)),
        ],
        out_shape=[
            jax.ShapeDtypeStruct((t, 1), jnp.int32),
            jax.ShapeDtypeStruct((t, 1), jnp.float32),
        ],
        scratch_shapes=[pltpu.VMEM((1, EPAD), jnp.float32)],
    )(xf, gwp, gbp)


def _ffn_body(x_ref, w2_ref, w3_ref, w1_ref, o_ref):
    x = x_ref[...]
    a = jnp.dot(x, w2_ref[0], preferred_element_type=jnp.float32)
    b = jnp.dot(x, w3_ref[0], preferred_element_type=jnp.float32)
    h = (a * lax.logistic(a)) * b
    o_ref[...] = jnp.dot(h, w1_ref[0], preferred_element_type=jnp.float32)


def _ffn(disp, w2, w3, w1, cap):
    rb = 128
    nrb = cap // rb
    rows = disp.shape[0]
    return pl.pallas_call(
        _ffn_body,
        grid=(E, nrb),
        in_specs=[
            pl.BlockSpec((rb, DIM), lambda e, r: (e * (cap // 128) + r, 0)),
            pl.BlockSpec((1, DIM, HIDDEN), lambda e, r: (e, 0, 0)),
            pl.BlockSpec((1, DIM, HIDDEN), lambda e, r: (e, 0, 0)),
            pl.BlockSpec((1, HIDDEN, DIM), lambda e, r: (e, 0, 0)),
        ],
        out_specs=pl.BlockSpec((rb, DIM), lambda e, r: (e * (cap // 128) + r, 0)),
        out_shape=jax.ShapeDtypeStruct((rows, DIM), jnp.float32),
    )(disp, w2, w3, w1)


def _scale_body(c_ref, g_ref, o_ref):
    g = g_ref[...]
    o_ref[...] = jnp.where(g > 0.0, g * c_ref[...], 0.0)


def _scale(comb, gwt):
    t = comb.shape[0]
    return pl.pallas_call(
        _scale_body,
        grid=(t // TB,),
        in_specs=[
            pl.BlockSpec((TB, DIM), lambda i: (i, 0)),
            pl.BlockSpec((TB, 1), lambda i: (i, 0)),
        ],
        out_specs=pl.BlockSpec((TB, DIM), lambda i: (i, 0)),
        out_shape=jax.ShapeDtypeStruct((t, DIM), jnp.float32),
    )(comb, gwt)


def kernel(x, gate_w, gate_b, w2, w3, w1, deterministic):
    b, s, d = x.shape
    t = b * s
    cap = int(t / E * CAPACITY_FACTOR)
    xf = x.reshape(t, d)
    gwp = jnp.zeros((d, EPAD), jnp.float32).at[:, :E].set(gate_w)
    gbp = jnp.zeros((1, EPAD), jnp.float32).at[0, :E].set(gate_b)

    slot2d, gwt = _routing(xf, gwp, gbp, cap)
    slot = slot2d.reshape(t)

    disp_rows = E * cap + 8
    inv = jnp.full((disp_rows,), t, jnp.int32).at[slot].set(
        jnp.arange(t, dtype=jnp.int32), mode="drop")
    xpad = jnp.concatenate([xf, jnp.zeros((1, d), jnp.float32)], axis=0)
    disp = xpad[inv]
    eout = _ffn(disp, w2, w3, w1, cap)
    comb = eout[slot]
    out = _scale(comb, gwt)
    return out.reshape(b, s, d)


# speedup vs baseline: 2.5264x; 1.3720x over previous
import functools

import jax
import jax.numpy as jnp
from jax import lax
from jax.experimental import pallas as pl
from jax.experimental.pallas import tpu as pltpu
from jax.experimental.pallas import tpu_sc as plsc

DIM = 1024
HIDDEN = 2048
E = 8
EPAD = 128
TB = 128
CAPACITY_FACTOR = 1.25


def _routing_body(cap, x_ref, gw_ref, gb_ref, slot_ref, gwt_ref, carry_ref):
    i = pl.program_id(0)

    @pl.when(i == 0)
    def _():
        carry_ref[...] = jnp.zeros_like(carry_ref)

    x = x_ref[...]
    logits = jnp.dot(x, gw_ref[...], preferred_element_type=jnp.float32)
    logits = logits + gb_ref[...]
    lane = lax.broadcasted_iota(jnp.int32, (TB, EPAD), 1)
    logits = jnp.where(lane < E, logits, -1e30)
    m = jnp.max(logits, axis=1, keepdims=True)
    eid = jnp.min(jnp.where(logits >= m, lane, EPAD), axis=1, keepdims=True)
    denom = jnp.sum(jnp.where(lane < E, jnp.exp(logits - m), 0.0),
                    axis=1, keepdims=True)
    gw_tok = 1.0 / denom

    onehot = (lane == eid).astype(jnp.float32)
    row = lax.broadcasted_iota(jnp.int32, (TB, TB), 0)
    col = lax.broadcasted_iota(jnp.int32, (TB, TB), 1)
    ltri = (col <= row).astype(jnp.float32)
    within = jnp.dot(ltri, onehot, preferred_element_type=jnp.float32)
    carry = carry_ref[...]
    pos = jnp.sum((within + carry) * onehot, axis=1, keepdims=True)
    carry_ref[...] = carry + jnp.sum(onehot, axis=0, keepdims=True)

    valid = pos <= cap
    slot = jnp.where(valid, eid * cap + pos.astype(jnp.int32) - 1, E * cap)
    slot_ref[...] = slot
    gwt_ref[...] = jnp.where(valid, gw_tok, 0.0)


def _routing(xf, gwp, gbp, cap):
    t = xf.shape[0]
    nb = t // TB
    return pl.pallas_call(
        functools.partial(_routing_body, cap),
        grid=(nb,),
        in_specs=[
            pl.BlockSpec((TB, DIM), lambda i: (i, 0)),
            pl.BlockSpec((DIM, EPAD), lambda i: (0, 0)),
            pl.BlockSpec((1, EPAD), lambda i: (0, 0)),
        ],
        out_specs=[
            pl.BlockSpec((TB, 1), lambda i: (i, 0)),
            pl.BlockSpec((TB, 1), lambda i: (i, 0)),
        ],
        out_shape=[
            jax.ShapeDtypeStruct((t, 1), jnp.int32),
            jax.ShapeDtypeStruct((t, 1), jnp.float32),
        ],
        scratch_shapes=[pltpu.VMEM((1, EPAD), jnp.float32)],
    )(xf, gwp, gbp)


def _ffn_body(x_ref, w2_ref, w3_ref, w1_ref, o_ref):
    x = x_ref[...]
    a = jnp.dot(x, w2_ref[0], preferred_element_type=jnp.float32)
    b = jnp.dot(x, w3_ref[0], preferred_element_type=jnp.float32)
    h = (a * lax.logistic(a)) * b
    o_ref[...] = jnp.dot(h, w1_ref[0], preferred_element_type=jnp.float32)


def _ffn(disp, w2, w3, w1, cap):
    rb = 128
    nrb = cap // rb
    rows = disp.shape[0]
    return pl.pallas_call(
        _ffn_body,
        grid=(E, nrb),
        in_specs=[
            pl.BlockSpec((rb, DIM), lambda e, r: (e * (cap // 128) + r, 0)),
            pl.BlockSpec((1, DIM, HIDDEN), lambda e, r: (e, 0, 0)),
            pl.BlockSpec((1, DIM, HIDDEN), lambda e, r: (e, 0, 0)),
            pl.BlockSpec((1, HIDDEN, DIM), lambda e, r: (e, 0, 0)),
        ],
        out_specs=pl.BlockSpec((rb, DIM), lambda e, r: (e * (cap // 128) + r, 0)),
        out_shape=jax.ShapeDtypeStruct((rows, DIM), jnp.float32),
    )(disp, w2, w3, w1)


_SC_CORES = 2
_SC_SUBCORES = 16
_NW = _SC_CORES * _SC_SUBCORES
_CHUNK = 32


def _sc_mesh():
    return plsc.VectorSubcoreMesh(core_axis_name="c", subcore_axis_name="s")


def _dispatch(xf, slot, disp_rows):
    t, d = xf.shape
    per_w = t // _NW

    @functools.partial(
        pl.kernel,
        mesh=_sc_mesh(),
        out_type=jax.ShapeDtypeStruct((disp_rows, d), jnp.float32),
        scratch_types=[
            pltpu.VMEM((_CHUNK,), jnp.int32),
            pltpu.VMEM((_CHUNK, d), jnp.float32),
        ],
    )
    def dispatch_kernel(x_hbm, slot_hbm, disp_hbm, idx_v, row_v):
        wid = lax.axis_index("s") * _SC_CORES + lax.axis_index("c")
        base = wid * per_w
        for c in range(per_w // _CHUNK):
            b = base + c * _CHUNK
            pltpu.sync_copy(slot_hbm.at[pl.ds(b, _CHUNK)], idx_v)
            pltpu.sync_copy(x_hbm.at[pl.ds(b, _CHUNK)], row_v)
            pltpu.sync_copy(row_v, disp_hbm.at[idx_v])

    return dispatch_kernel(xf, slot)


def _combine(eout, slot, t):
    d = eout.shape[1]
    per_w = t // _NW

    @functools.partial(
        pl.kernel,
        mesh=_sc_mesh(),
        out_type=jax.ShapeDtypeStruct((t, d), jnp.float32),
        scratch_types=[
            pltpu.VMEM((_CHUNK,), jnp.int32),
            pltpu.VMEM((_CHUNK, d), jnp.float32),
        ],
    )
    def combine_kernel(eout_hbm, slot_hbm, comb_hbm, idx_v, row_v):
        wid = lax.axis_index("s") * _SC_CORES + lax.axis_index("c")
        base = wid * per_w
        for c in range(per_w // _CHUNK):
            b = base + c * _CHUNK
            pltpu.sync_copy(slot_hbm.at[pl.ds(b, _CHUNK)], idx_v)
            pltpu.sync_copy(eout_hbm.at[idx_v], row_v)
            pltpu.sync_copy(row_v, comb_hbm.at[pl.ds(b, _CHUNK)])

    return combine_kernel(eout, slot)


def _scale_body(c_ref, g_ref, o_ref):
    g = g_ref[...]
    o_ref[...] = jnp.where(g > 0.0, g * c_ref[...], 0.0)


def _scale(comb, gwt):
    t = comb.shape[0]
    return pl.pallas_call(
        _scale_body,
        grid=(t // TB,),
        in_specs=[
            pl.BlockSpec((TB, DIM), lambda i: (i, 0)),
            pl.BlockSpec((TB, 1), lambda i: (i, 0)),
        ],
        out_specs=pl.BlockSpec((TB, DIM), lambda i: (i, 0)),
        out_shape=jax.ShapeDtypeStruct((t, DIM), jnp.float32),
    )(comb, gwt)


def kernel(x, gate_w, gate_b, w2, w3, w1, deterministic):
    b, s, d = x.shape
    t = b * s
    cap = int(t / E * CAPACITY_FACTOR)
    xf = x.reshape(t, d)
    gwp = jnp.zeros((d, EPAD), jnp.float32).at[:, :E].set(gate_w)
    gbp = jnp.zeros((1, EPAD), jnp.float32).at[0, :E].set(gate_b)

    slot2d, gwt = _routing(xf, gwp, gbp, cap)
    slot = slot2d.reshape(t)

    disp_rows = E * cap + 8
    disp = _dispatch(xf, slot, disp_rows)
    eout = _ffn(disp, w2, w3, w1, cap)
    comb = _combine(eout, slot, t)
    out = _scale(comb, gwt)
    return out.reshape(b, s, d)


# speedup vs baseline: 3.3313x; 1.3186x over previous
import functools

import jax
import jax.numpy as jnp
from jax import lax
from jax.experimental import pallas as pl
from jax.experimental.pallas import tpu as pltpu
from jax.experimental.pallas import tpu_sc as plsc

DIM = 1024
HIDDEN = 2048
E = 8
EPAD = 128
TB = 256
CAPACITY_FACTOR = 1.25


def _routing_body(cap, x_ref, gw_ref, gb_ref, slot_ref, gwt_ref, carry_ref):
    i = pl.program_id(0)

    @pl.when(i == 0)
    def _():
        carry_ref[...] = jnp.zeros_like(carry_ref)

    x = x_ref[...]
    logits = jnp.dot(x, gw_ref[...], preferred_element_type=jnp.float32)
    logits = logits + gb_ref[...]
    lane = lax.broadcasted_iota(jnp.int32, (TB, EPAD), 1)
    logits = jnp.where(lane < E, logits, -1e30)
    m = jnp.max(logits, axis=1, keepdims=True)
    eid = jnp.min(jnp.where(logits >= m, lane, EPAD), axis=1, keepdims=True)
    denom = jnp.sum(jnp.where(lane < E, jnp.exp(logits - m), 0.0),
                    axis=1, keepdims=True)
    gw_tok = 1.0 / denom

    onehot = (lane == eid).astype(jnp.float32)
    row = lax.broadcasted_iota(jnp.int32, (TB, TB), 0)
    col = lax.broadcasted_iota(jnp.int32, (TB, TB), 1)
    ltri = (col <= row).astype(jnp.float32)
    within = jnp.dot(ltri, onehot, preferred_element_type=jnp.float32)
    carry = carry_ref[...]
    pos = jnp.sum((within + carry) * onehot, axis=1, keepdims=True)
    carry_ref[...] = carry + jnp.sum(onehot, axis=0, keepdims=True)

    valid = pos <= cap
    slot = jnp.where(valid, eid * cap + pos.astype(jnp.int32) - 1, E * cap)
    slot_ref[...] = slot
    gwt_ref[...] = jnp.where(valid, gw_tok, 0.0)


def _routing(xf, gwp, gbp, cap):
    t = xf.shape[0]
    nb = t // TB
    return pl.pallas_call(
        functools.partial(_routing_body, cap),
        grid=(nb,),
        in_specs=[
            pl.BlockSpec((TB, DIM), lambda i: (i, 0)),
            pl.BlockSpec((DIM, EPAD), lambda i: (0, 0)),
            pl.BlockSpec((1, EPAD), lambda i: (0, 0)),
        ],
        out_specs=[
            pl.BlockSpec((TB, 1), lambda i: (i, 0)),
            pl.BlockSpec((TB, 1), lambda i: (i, 0)),
        ],
        out_shape=[
            jax.ShapeDtypeStruct((t, 1), jnp.int32),
            jax.ShapeDtypeStruct((t, 1), jnp.float32),
        ],
        scratch_shapes=[pltpu.VMEM((1, EPAD), jnp.float32)],
    )(xf, gwp, gbp)


_NH = 4
_HB = HIDDEN // _NH


def _ffn_body(x_ref, w2_ref, w3_ref, w1_ref, o_ref):
    hb = pl.program_id(1)
    x = x_ref[...]
    a = jnp.dot(x, w2_ref[0], preferred_element_type=jnp.float32)
    b = jnp.dot(x, w3_ref[0], preferred_element_type=jnp.float32)
    h = (a * lax.logistic(a)) * b
    p = jnp.dot(h, w1_ref[0], preferred_element_type=jnp.float32)

    @pl.when(hb == 0)
    def _():
        o_ref[...] = p

    @pl.when(hb != 0)
    def _():
        o_ref[...] += p


def _ffn(disp, w2, w3, w1, cap):
    rows = disp.shape[0]
    return pl.pallas_call(
        _ffn_body,
        grid=(E, _NH),
        in_specs=[
            pl.BlockSpec((cap, DIM), lambda e, h: (e, 0)),
            pl.BlockSpec((1, DIM, _HB), lambda e, h: (e, 0, h)),
            pl.BlockSpec((1, DIM, _HB), lambda e, h: (e, 0, h)),
            pl.BlockSpec((1, _HB, DIM), lambda e, h: (e, h, 0)),
        ],
        out_specs=pl.BlockSpec((cap, DIM), lambda e, h: (e, 0)),
        out_shape=jax.ShapeDtypeStruct((rows, DIM), jnp.float32),
    )(disp, w2, w3, w1)


_SC_CORES = 2
_SC_SUBCORES = 16
_NW = _SC_CORES * _SC_SUBCORES
_CHUNK = 32


def _sc_mesh():
    return plsc.VectorSubcoreMesh(core_axis_name="c", subcore_axis_name="s")


def _dispatch(xf, slot, disp_rows):
    t, d = xf.shape
    per_w = t // _NW

    @functools.partial(
        pl.kernel,
        mesh=_sc_mesh(),
        out_type=jax.ShapeDtypeStruct((disp_rows, d), jnp.float32),
        scratch_types=[
            pltpu.VMEM((_CHUNK,), jnp.int32),
            pltpu.VMEM((_CHUNK, d), jnp.float32),
        ],
    )
    def dispatch_kernel(x_hbm, slot_hbm, disp_hbm, idx_v, row_v):
        wid = lax.axis_index("s") * _SC_CORES + lax.axis_index("c")
        base = wid * per_w
        for c in range(per_w // _CHUNK):
            b = base + c * _CHUNK
            pltpu.sync_copy(slot_hbm.at[pl.ds(b, _CHUNK)], idx_v)
            pltpu.sync_copy(x_hbm.at[pl.ds(b, _CHUNK)], row_v)
            pltpu.sync_copy(row_v, disp_hbm.at[idx_v])

    return dispatch_kernel(xf, slot)


def _combine(eout, slot, t):
    d = eout.shape[1]
    per_w = t // _NW

    @functools.partial(
        pl.kernel,
        mesh=_sc_mesh(),
        out_type=jax.ShapeDtypeStruct((t, d), jnp.float32),
        scratch_types=[
            pltpu.VMEM((_CHUNK,), jnp.int32),
            pltpu.VMEM((_CHUNK, d), jnp.float32),
        ],
    )
    def combine_kernel(eout_hbm, slot_hbm, comb_hbm, idx_v, row_v):
        wid = lax.axis_index("s") * _SC_CORES + lax.axis_index("c")
        base = wid * per_w
        for c in range(per_w // _CHUNK):
            b = base + c * _CHUNK
            pltpu.sync_copy(slot_hbm.at[pl.ds(b, _CHUNK)], idx_v)
            pltpu.sync_copy(eout_hbm.at[idx_v], row_v)
            pltpu.sync_copy(row_v, comb_hbm.at[pl.ds(b, _CHUNK)])

    return combine_kernel(eout, slot)


def _scale_body(c_ref, g_ref, o_ref):
    g = g_ref[...]
    o_ref[...] = jnp.where(g > 0.0, g * c_ref[...], 0.0)


def _scale(comb, gwt):
    t = comb.shape[0]
    sb = 512
    return pl.pallas_call(
        _scale_body,
        grid=(t // sb,),
        in_specs=[
            pl.BlockSpec((sb, DIM), lambda i: (i, 0)),
            pl.BlockSpec((sb, 1), lambda i: (i, 0)),
        ],
        out_specs=pl.BlockSpec((sb, DIM), lambda i: (i, 0)),
        out_shape=jax.ShapeDtypeStruct((t, DIM), jnp.float32),
    )(comb, gwt)


def kernel(x, gate_w, gate_b, w2, w3, w1, deterministic):
    b, s, d = x.shape
    t = b * s
    cap = int(t / E * CAPACITY_FACTOR)
    xf = x.reshape(t, d)
    gwp = jnp.zeros((d, EPAD), jnp.float32).at[:, :E].set(gate_w)
    gbp = jnp.zeros((1, EPAD), jnp.float32).at[0, :E].set(gate_b)

    slot2d, gwt = _routing(xf, gwp, gbp, cap)
    slot = slot2d.reshape(t)

    disp_rows = E * cap + 8
    disp = _dispatch(xf, slot, disp_rows)
    eout = _ffn(disp, w2, w3, w1, cap)
    comb = _combine(eout, slot, t)
    out = _scale(comb, gwt)
    return out.reshape(b, s, d)
